# baseline (speedup 1.0000x reference)
import jax
import jax.numpy as jnp
from jax.experimental import pallas as pl

N = 8192
TOP_K = 32
INF = 1e20


def kernel(node_features, W):
    context_fc = jax.nn.relu(node_features @ W.T)
    attention = context_fc @ context_fc.T
    knn_val, knn_ind = jax.lax.top_k(attention, TOP_K)
    rows = jnp.arange(attention.shape[0])[:, None]
    adj = jnp.full(attention.shape, -INF, dtype=attention.dtype)
    adj = adj.at[rows, knn_ind].set(knn_val)
    adj = jax.nn.softmax(adj, axis=-1)
    return (node_features, adj)


# speedup vs baseline: 3.3971x; 3.3971x over previous
import jax
import jax.numpy as jnp
from jax import lax
from jax.experimental import pallas as pl
from jax.experimental.pallas import tpu as pltpu

_BLK = 128
_K = 32


def _ctx_kernel(x_ref, w_ref, c_ref):
    c_ref[...] = jax.nn.relu(
        lax.dot_general(x_ref[...], w_ref[...], (((1,), (1,)), ((), ())),
                        preferred_element_type=jnp.float32))


def _stats_kernel(c_full_ref, c_blk_ref, t_ref, m_ref, z_ref, s_ref):
    s_ref[...] = lax.dot_general(
        c_full_ref[...], c_blk_ref[...], (((1,), (1,)), ((), ())),
        preferred_element_type=jnp.float32)
    zero = jnp.zeros((1, _BLK), jnp.float32)

    def body(_, carry):
        big_m, z, t, kc = carry
        sw = s_ref[...]
        m = jnp.max(sw, axis=0, keepdims=True)
        active = kc < float(_K)
        big_m = jnp.where(kc == 0.0, m, big_m)
        eq = sw == m
        cnt = jnp.sum(eq.astype(jnp.float32), axis=0, keepdims=True)
        s_ref[...] = jnp.where(eq & active, -1.0, sw)
        z = jnp.where(active, z + cnt * jnp.exp(m - big_m), z)
        t = jnp.where(active, m, t)
        kc = jnp.where(active, kc + cnt, kc)
        return big_m, z, t, kc

    big_m, z, t, _ = lax.fori_loop(0, _K, body, (zero, zero, zero, zero))
    t_ref[...] = t.reshape(1, 1, _BLK)
    m_ref[...] = big_m.reshape(1, 1, _BLK)
    z_ref[...] = z.reshape(1, 1, _BLK)


def _adj_kernel(c_full_ref, c_blk_ref, t_ref, m_ref, z_ref, o_ref):
    s = lax.dot_general(
        c_full_ref[...], c_blk_ref[...], (((1,), (1,)), ((), ())),
        preferred_element_type=jnp.float32)
    t = t_ref[...]
    big_m = m_ref[...]
    invz = 1.0 / z_ref[...]
    o_ref[...] = jnp.where(s >= t, jnp.exp(s - big_m) * invz, 0.0)


def _build(n, d, h, interpret=False):
    nb = n // _BLK

    ctx = pl.pallas_call(
        _ctx_kernel,
        out_shape=jax.ShapeDtypeStruct((n, h), jnp.float32),
        interpret=interpret,
    )

    stats = pl.pallas_call(
        _stats_kernel,
        grid=(nb,),
        in_specs=[
            pl.BlockSpec((n, h), lambda j: (0, 0)),
            pl.BlockSpec((_BLK, h), lambda j: (j, 0)),
        ],
        out_specs=[
            pl.BlockSpec((1, 1, _BLK), lambda j: (j, 0, 0)),
            pl.BlockSpec((1, 1, _BLK), lambda j: (j, 0, 0)),
            pl.BlockSpec((1, 1, _BLK), lambda j: (j, 0, 0)),
        ],
        out_shape=[jax.ShapeDtypeStruct((nb, 1, _BLK), jnp.float32)] * 3,
        scratch_shapes=[pltpu.VMEM((n, _BLK), jnp.float32)],
        compiler_params=pltpu.CompilerParams(
            dimension_semantics=("arbitrary",)),
        interpret=interpret,
    )

    adj_call = pl.pallas_call(
        _adj_kernel,
        grid=(nb,),
        in_specs=[
            pl.BlockSpec((n, h), lambda j: (0, 0)),
            pl.BlockSpec((_BLK, h), lambda j: (j, 0)),
            pl.BlockSpec((n, 1), lambda j: (0, 0)),
            pl.BlockSpec((n, 1), lambda j: (0, 0)),
            pl.BlockSpec((n, 1), lambda j: (0, 0)),
        ],
        out_specs=pl.BlockSpec((n, _BLK), lambda j: (0, j)),
        out_shape=jax.ShapeDtypeStruct((n, n), jnp.float32),
        compiler_params=pltpu.CompilerParams(
            dimension_semantics=("arbitrary",)),
        interpret=interpret,
    )
    return ctx, stats, adj_call


def _run(node_features, W, interpret=False):
    n, d = node_features.shape
    h = W.shape[0]
    ctx, stats, adj_call = _build(n, d, h, interpret)
    c = ctx(node_features, W)
    t, big_m, z = stats(c, c)
    tv = t.reshape(n, 1)
    mv = big_m.reshape(n, 1)
    zv = z.reshape(n, 1)
    adj = adj_call(c, c, tv, mv, zv)
    return (node_features, adj)


def kernel(node_features, W):
    return _run(node_features, W, interpret=False)


# speedup vs baseline: 45.2560x; 13.3219x over previous
import jax
import jax.numpy as jnp
from jax import lax
from jax.experimental import pallas as pl
from jax.experimental.pallas import tpu as pltpu

_BLK = 128
_K = 32
_LOG2E = 1.4426950408889634


def _ctx_kernel(x_ref, w_ref, c_ref):
    c_ref[...] = jax.nn.relu(
        lax.dot_general(x_ref[...], w_ref[...], (((1,), (1,)), ((), ())),
                        preferred_element_type=jnp.float32))


def _top32_desc(s):
    n, b = s.shape
    lists = n // _K
    a = s.reshape(_K, lists, b)
    for sz in (2, 4, 8, 16, 32):
        st = sz // 2
        while st >= 1:
            g = _K // (2 * st)
            a4 = a.reshape(g, 2, st, lists, b)
            x, y = a4[:, 0], a4[:, 1]
            mx = jnp.maximum(x, y)
            mn = jnp.minimum(x, y)
            if sz == _K:
                first, second = mx, mn
            else:
                q = lax.broadcasted_iota(jnp.int32, (g, 1, 1, 1), 0)
                dm = ((q * (2 * st)) // sz) % 2 == 0
                first = jnp.where(dm, mx, mn)
                second = jnp.where(dm, mn, mx)
            a = jnp.stack([first, second], axis=1).reshape(_K, lists, b)
            st //= 2
    cur = lists
    while cur > 1:
        h = cur // 2
        x = a[:, :h]
        y = a[:, h:]
        yr = jnp.stack([y[_K - 1 - k] for k in range(_K)], axis=0)
        a = jnp.maximum(x, yr)
        for st in (16, 8, 4, 2, 1):
            g = _K // (2 * st)
            a4 = a.reshape(g, 2, st, h, b)
            x2, y2 = a4[:, 0], a4[:, 1]
            a = jnp.stack([jnp.maximum(x2, y2), jnp.minimum(x2, y2)],
                          axis=1).reshape(_K, h, b)
        cur = h
    return a.reshape(_K, b)


def _stats_kernel(c_full_ref, c_blk_ref, t_ref, a_ref):
    s = lax.dot_general(
        c_full_ref[...], c_blk_ref[...], (((1,), (1,)), ((), ())),
        preferred_element_type=jnp.float32)
    vals = _top32_desc(s)
    big_m = vals[0:1, :]
    t = vals[_K - 1:_K, :]
    z = jnp.sum(jnp.exp(vals - big_m), axis=0, keepdims=True)
    coef = big_m * _LOG2E + jnp.log2(z)
    t_ref[...] = t.reshape(1, 1, _BLK)
    a_ref[...] = coef.reshape(1, 1, _BLK)


def _adj_kernel(c_full_ref, c_blk_ref, tb_ref, ab_ref, o_ref):
    s = lax.dot_general(
        c_full_ref[...], c_blk_ref[...], (((1,), (1,)), ((), ())),
        preferred_element_type=jnp.float32)
    o_ref[...] = jnp.where(
        s >= tb_ref[...],
        jnp.exp2(s * _LOG2E - ab_ref[...]),
        0.0)


def _build(n, d, h, interpret=False):
    nb = n // _BLK

    ctx = pl.pallas_call(
        _ctx_kernel,
        out_shape=jax.ShapeDtypeStruct((n, h), jnp.float32),
        interpret=interpret,
    )

    stats = pl.pallas_call(
        _stats_kernel,
        grid=(nb,),
        in_specs=[
            pl.BlockSpec((n, h), lambda j: (0, 0)),
            pl.BlockSpec((_BLK, h), lambda j: (j, 0)),
        ],
        out_specs=[
            pl.BlockSpec((1, 1, _BLK), lambda j: (j, 0, 0)),
            pl.BlockSpec((1, 1, _BLK), lambda j: (j, 0, 0)),
        ],
        out_shape=[jax.ShapeDtypeStruct((nb, 1, _BLK), jnp.float32)] * 2,
        compiler_params=pltpu.CompilerParams(
            dimension_semantics=("arbitrary",)),
        interpret=interpret,
    )

    adj_call = pl.pallas_call(
        _adj_kernel,
        grid=(nb,),
        in_specs=[
            pl.BlockSpec((n, h), lambda j: (0, 0)),
            pl.BlockSpec((_BLK, h), lambda j: (j, 0)),
            pl.BlockSpec((n, _BLK), lambda j: (0, 0)),
            pl.BlockSpec((n, _BLK), lambda j: (0, 0)),
        ],
        out_specs=pl.BlockSpec((n, _BLK), lambda j: (0, j)),
        out_shape=jax.ShapeDtypeStruct((n, n), jnp.float32),
        compiler_params=pltpu.CompilerParams(
            dimension_semantics=("arbitrary",)),
        interpret=interpret,
    )
    return ctx, stats, adj_call


def _run(node_features, W, interpret=False):
    n, d = node_features.shape
    h = W.shape[0]
    ctx, stats, adj_call = _build(n, d, h, interpret)
    c = ctx(node_features, W)
    t, coef = stats(c, c)
    tb = jnp.broadcast_to(t.reshape(n, 1), (n, _BLK))
    ab = jnp.broadcast_to(coef.reshape(n, 1), (n, _BLK))
    adj = adj_call(c, c, tb, ab)
    return (node_features, adj)


def kernel(node_features, W):
    return _run(node_features, W, interpret=False)


# speedup vs baseline: 46.2104x; 1.0211x over previous
import jax
import jax.numpy as jnp
from jax import lax
from jax.experimental import pallas as pl
from jax.experimental.pallas import tpu as pltpu

_BLK = 128
_K = 32
_LOG2E = 1.4426950408889634


def _top32_desc(s):
    n, b = s.shape
    lists = n // _K
    a = s.reshape(_K, lists, b)
    for sz in (2, 4, 8, 16, 32):
        st = sz // 2
        while st >= 1:
            g = _K // (2 * st)
            a4 = a.reshape(g, 2, st, lists, b)
            x, y = a4[:, 0], a4[:, 1]
            mx = jnp.maximum(x, y)
            mn = jnp.minimum(x, y)
            if sz == _K:
                first, second = mx, mn
            else:
                q = lax.broadcasted_iota(jnp.int32, (g, 1, 1, 1), 0)
                dm = ((q * (2 * st)) // sz) % 2 == 0
                first = jnp.where(dm, mx, mn)
                second = jnp.where(dm, mn, mx)
            a = jnp.stack([first, second], axis=1).reshape(_K, lists, b)
            st //= 2
    cur = lists
    while cur > 1:
        h = cur // 2
        x = a[:, :h]
        y = a[:, h:]
        yr = jnp.stack([y[_K - 1 - k] for k in range(_K)], axis=0)
        a = jnp.maximum(x, yr)
        for st in (16, 8, 4, 2, 1):
            g = _K // (2 * st)
            a4 = a.reshape(g, 2, st, h, b)
            x2, y2 = a4[:, 0], a4[:, 1]
            a = jnp.stack([jnp.maximum(x2, y2), jnp.minimum(x2, y2)],
                          axis=1).reshape(_K, h, b)
        cur = h
    return a.reshape(_K, b)


def _stats_kernel(x_ref, w_ref, c_ref, tb_ref, ab_ref, c_vmem):
    j = pl.program_id(0)

    @pl.when(j == 0)
    def _():
        c = jax.nn.relu(
            lax.dot_general(x_ref[...], w_ref[...], (((1,), (1,)), ((), ())),
                            preferred_element_type=jnp.float32))
        c_vmem[...] = c
        c_ref[...] = c

    c_blk = c_vmem[pl.ds(j * _BLK, _BLK), :]
    s = lax.dot_general(
        c_vmem[...], c_blk, (((1,), (1,)), ((), ())),
        preferred_element_type=jnp.float32)
    vals = _top32_desc(s)
    big_m = vals[0:1, :]
    t = vals[_K - 1:_K, :]
    z = jnp.sum(jnp.exp(vals - big_m), axis=0, keepdims=True)
    coef = big_m * _LOG2E + jnp.log2(z)
    tb_ref[...] = jnp.broadcast_to(t.reshape(_BLK, 1), (_BLK, _BLK))
    ab_ref[...] = jnp.broadcast_to(coef.reshape(_BLK, 1), (_BLK, _BLK))


def _adj_kernel(c_full_ref, c_blk_ref, tb_ref, ab_ref, o_ref):
    s = lax.dot_general(
        c_full_ref[...], c_blk_ref[...], (((1,), (1,)), ((), ())),
        preferred_element_type=jnp.float32)
    o_ref[...] = jnp.where(
        s >= tb_ref[...],
        jnp.exp2(s * _LOG2E - ab_ref[...]),
        0.0)


def _build(n, d, h, interpret=False):
    nb = n // _BLK

    stats = pl.pallas_call(
        _stats_kernel,
        grid=(nb,),
        in_specs=[
            pl.BlockSpec((n, d), lambda j: (0, 0)),
            pl.BlockSpec((h, d), lambda j: (0, 0)),
        ],
        out_specs=[
            pl.BlockSpec((n, h), lambda j: (0, 0)),
            pl.BlockSpec((_BLK, _BLK), lambda j: (j, 0)),
            pl.BlockSpec((_BLK, _BLK), lambda j: (j, 0)),
        ],
        out_shape=[
            jax.ShapeDtypeStruct((n, h), jnp.float32),
            jax.ShapeDtypeStruct((n, _BLK), jnp.float32),
            jax.ShapeDtypeStruct((n, _BLK), jnp.float32),
        ],
        scratch_shapes=[pltpu.VMEM((n, h), jnp.float32)],
        compiler_params=pltpu.CompilerParams(
            dimension_semantics=("arbitrary",)),
        interpret=interpret,
    )

    adj_call = pl.pallas_call(
        _adj_kernel,
        grid=(nb,),
        in_specs=[
            pl.BlockSpec((n, h), lambda j: (0, 0)),
            pl.BlockSpec((_BLK, h), lambda j: (j, 0)),
            pl.BlockSpec((n, _BLK), lambda j: (0, 0)),
            pl.BlockSpec((n, _BLK), lambda j: (0, 0)),
        ],
        out_specs=pl.BlockSpec((n, _BLK), lambda j: (0, j)),
        out_shape=jax.ShapeDtypeStruct((n, n), jnp.float32),
        compiler_params=pltpu.CompilerParams(
            dimension_semantics=("arbitrary",)),
        interpret=interpret,
    )
    return stats, adj_call


def _run(node_features, W, interpret=False):
    n, d = node_features.shape
    h = W.shape[0]
    stats, adj_call = _build(n, d, h, interpret)
    c, tb, ab = stats(node_features, W)
    adj = adj_call(c, c, tb, ab)
    return (node_features, adj)


def kernel(node_features, W):
    return _run(node_features, W, interpret=False)
